# baseline (speedup 1.0000x reference)
import jax
import jax.numpy as jnp
from jax.experimental import pallas as pl
from jax.experimental.pallas import tpu as pltpu

N_LAYERS_ = 10
BATCH_ = 32
CHANNELS_ = 1024
L_CACHE_ = 20
LAYER_IDX_ = 0
NCHUNK_ = 10
CPC_ = L_CACHE_ // NCHUNK_


def _conv_kernel(cp_ref, a_hbm, wt_ref, bx_ref, out_ref, a_s, sems):
    for ch in range(NCHUNK_):
        pltpu.make_async_copy(
            a_hbm.at[LAYER_IDX_, pl.ds(ch * CPC_, CPC_)],
            a_s.at[pl.ds(ch * CPC_, CPC_)],
            sems.at[ch],
        ).start()
    cp = jnp.clip(cp_ref[0], 0, L_CACHE_ - 1)
    cpp = jax.lax.rem(cp + 1, L_CACHE_)
    wt = wt_ref[...]
    taps = jax.lax.broadcasted_iota(jnp.int32, (L_CACHE_, 1, 1), 0)
    wcp = jnp.sum(jnp.where(taps == cp, wt, 0.0), axis=0)
    bx = jnp.reshape(bx_ref[...], (BATCH_, CHANNELS_))
    acc = bx * wcp
    zrow = jnp.zeros((1, CHANNELS_), jnp.float32)
    for ch in range(NCHUNK_):
        pltpu.make_async_copy(
            a_hbm.at[LAYER_IDX_, pl.ds(ch * CPC_, CPC_)],
            a_s.at[pl.ds(ch * CPC_, CPC_)],
            sems.at[ch],
        ).wait()
        for k in range(CPC_):
            m = ch * CPC_ + k
            row = jnp.where(m == cpp, zrow, wt[(m + L_CACHE_ - 1) % L_CACHE_])
            acc = acc + a_s[m] * row
    out_ref[...] = jnp.reshape(acc, (BATCH_, CHANNELS_ // 128, 128))


def kernel(Bx, cache_position, seq_len, conv_cache, conv_weight):
    del seq_len
    at = jnp.transpose(conv_cache, (0, 3, 1, 2))
    wt = jnp.transpose(conv_weight, (1, 0))[:, None, :]
    bx = jnp.reshape(Bx, (BATCH_, CHANNELS_ // 128, 128))
    out = pl.pallas_call(
        _conv_kernel,
        in_specs=[
            pl.BlockSpec(memory_space=pltpu.MemorySpace.SMEM),
            pl.BlockSpec(memory_space=pltpu.MemorySpace.HBM),
            pl.BlockSpec((L_CACHE_, 1, CHANNELS_), lambda: (0, 0, 0)),
            pl.BlockSpec((BATCH_, CHANNELS_ // 128, 128),
                         lambda: (0, 0, 0)),
        ],
        out_specs=pl.BlockSpec((BATCH_, CHANNELS_ // 128, 128),
                               lambda: (0, 0, 0)),
        scratch_shapes=[
            pltpu.VMEM((L_CACHE_, BATCH_, CHANNELS_), jnp.float32),
            pltpu.SemaphoreType.DMA((NCHUNK_,)),
        ],
        out_shape=jax.ShapeDtypeStruct((BATCH_, CHANNELS_ // 128, 128),
                                       jnp.float32),
    )(cache_position, at, wt, bx)
    return out.reshape(BATCH_, CHANNELS_, 1)


# speedup vs baseline: 1.3693x; 1.3693x over previous
import jax
import jax.numpy as jnp
from jax.experimental import pallas as pl
from jax.experimental.pallas import tpu as pltpu

N_LAYERS_ = 10
BATCH_ = 32
CHANNELS_ = 1024
L_CACHE_ = 20
LAYER_IDX_ = 0
NCHUNK_ = 10
CPC_ = L_CACHE_ // NCHUNK_


def _conv_kernel(cp_ref, a_hbm, wt_ref, bx_ref, out_ref, a_s, sems):
    for ch in range(NCHUNK_):
        pltpu.make_async_copy(
            a_hbm.at[LAYER_IDX_, pl.ds(ch * CPC_, CPC_)],
            a_s.at[pl.ds(ch * CPC_, CPC_)],
            sems.at[ch],
        ).start()
    cp = jnp.clip(cp_ref[0], 0, L_CACHE_ - 1)
    cpp = jax.lax.rem(cp + 1, L_CACHE_)
    wt = wt_ref[...]
    taps = jax.lax.broadcasted_iota(jnp.int32, (L_CACHE_, 1), 0)
    wcp = jnp.sum(jnp.where(taps == cp, wt, 0.0), axis=0,
                  keepdims=True)
    bx = jnp.reshape(bx_ref[...], (BATCH_, CHANNELS_))
    acc = bx * wcp
    zrow = jnp.zeros((1, CHANNELS_), jnp.float32)
    for ch in range(NCHUNK_):
        pltpu.make_async_copy(
            a_hbm.at[LAYER_IDX_, pl.ds(ch * CPC_, CPC_)],
            a_s.at[pl.ds(ch * CPC_, CPC_)],
            sems.at[ch],
        ).wait()
        for k in range(CPC_):
            m = ch * CPC_ + k
            mw = (m + L_CACHE_ - 1) % L_CACHE_
            row = jnp.where(m == cpp, zrow, wt[mw:mw + 1])
            acc = acc + a_s[m] * row
    out_ref[...] = jnp.reshape(acc, (BATCH_, CHANNELS_ // 128, 128))


def kernel(Bx, cache_position, seq_len, conv_cache, conv_weight):
    del seq_len
    at = jnp.transpose(conv_cache, (0, 3, 1, 2))
    wt = jnp.transpose(conv_weight, (1, 0))
    bx = jnp.reshape(Bx, (BATCH_, CHANNELS_ // 128, 128))
    out = pl.pallas_call(
        _conv_kernel,
        in_specs=[
            pl.BlockSpec(memory_space=pltpu.MemorySpace.SMEM),
            pl.BlockSpec(memory_space=pltpu.MemorySpace.HBM),
            pl.BlockSpec((L_CACHE_, CHANNELS_), lambda: (0, 0)),
            pl.BlockSpec((BATCH_, CHANNELS_ // 128, 128),
                         lambda: (0, 0, 0)),
        ],
        out_specs=pl.BlockSpec((BATCH_, CHANNELS_ // 128, 128),
                               lambda: (0, 0, 0)),
        scratch_shapes=[
            pltpu.VMEM((L_CACHE_, BATCH_, CHANNELS_), jnp.float32),
            pltpu.SemaphoreType.DMA((NCHUNK_,)),
        ],
        out_shape=jax.ShapeDtypeStruct((BATCH_, CHANNELS_ // 128, 128),
                                       jnp.float32),
    )(cache_position, at, wt, bx)
    return out.reshape(BATCH_, CHANNELS_, 1)
